# baseline (speedup 1.0000x reference)
import functools

import jax
import jax.numpy as jnp
from jax import lax
from jax.experimental import pallas as pl
from jax.experimental.pallas import tpu as pltpu
from jax.experimental.pallas import tpu_sc as plsc

B, N, NPOINT, K = 4, 8192, 2048, 32
IN_CH, OUT_CH = 128, 256
PART, RADIUS = 5, 2.0
G = PART * PART + 1
GU = 20
P = B * NPOINT
PG = 2
NGRP = P // PG
GW = PG * K
CH = GW // 16
NW = 32
NH = 2
HP = P // NH
HNGRP = HP // PG
GRP_W = HNGRP // NW
XROWS = HP // GW
XROW_W = XROWS // NW


RB = 4
GB = 4


def _sc_kernel_body(h, tbl, nbrr, lcx, lcy, lcz, didx, xyztbl,
                    feat, nxyz,
                    didx_v, nbr_v, sbin_v,
                    lcx_v, lcy_v, lcz_v, rows_v, zero_v,
                    grid_sh, sem, gsem, zsem, osem):
    cid = lax.axis_index("c")
    sid = lax.axis_index("s")
    wid = cid * 16 + sid
    bn = ((h * HP + wid * (HP // NW)) // NPOINT) * N
    iota = lax.iota(jnp.int32, 16)

    pltpu.sync_copy(didx.at[pl.ds(wid * XROW_W, XROW_W)], didx_v)
    for r in range(XROW_W):
        row = wid * XROW_W + r
        for c in range(GW // 16):
            sl = pl.ds(c * 16, 16)
            didx_v[r, sl] = didx_v[r, sl] + bn
        pltpu.async_copy(xyztbl.at[didx_v.at[r]], rows_v.at[0], sem).wait()
        pltpu.sync_copy(rows_v.at[0], nxyz.at[row])

    g0 = wid * GRP_W
    pltpu.sync_copy(nbrr.at[pl.ds(g0, GRP_W)], nbr_v)
    pltpu.sync_copy(lcx.at[pl.ds(g0, GRP_W)], lcx_v)
    pltpu.sync_copy(lcy.at[pl.ds(g0, GRP_W)], lcy_v)
    pltpu.sync_copy(lcz.at[pl.ds(g0, GRP_W)], lcz_v)

    zv = jnp.zeros((16,), jnp.float32)

    def _zero_body(i, _):
        for c in range(8):
            zero_v[i, pl.ds(c * 16, 16)] = zv
        return 0

    lax.fori_loop(0, PG * GU, _zero_body, 0)

    def _idx_body(g, _):
        for c in range(CH):
            sl = pl.ds(c * 16, 16)
            nbr_v[g, sl] = nbr_v[g, sl] + bn
            x = lcx_v[g, sl]
            y = lcy_v[g, sl]
            fx = (x + RADIUS) / (2.0 * RADIUS) * (PART - 1 - 0.01)
            fy = (y + RADIUS) / (2.0 * RADIUS) * (PART - 1 - 0.01)
            bidx = fx.astype(jnp.int32) * PART + fy.astype(jnp.int32) + 1
            filt = ((x * x > RADIUS * RADIUS) |
                    (y * y > RADIUS * RADIUS))
            sbin_v[g, sl] = jnp.where(filt, 0, bidx) + (c // 2) * GU
        return 0

    lax.fori_loop(0, GRP_W, _idx_body, 0)

    for r in range(GB):
        pltpu.async_copy(zero_v, grid_sh.at[sid, r], zsem.at[r])
    pltpu.async_copy(tbl.at[nbr_v.at[0]], rows_v.at[0], gsem.at[0])

    def _quad_body(go, _):
        for rr in range(GB):
            g = go * GB + rr
            rb = rr % RB
            rowbuf = rows_v.at[rb]
            pltpu.make_async_copy(tbl.at[nbr_v.at[g]], rowbuf,
                                  gsem.at[rb]).wait()

            @pl.when(g + 1 < GRP_W)
            def _():
                nb = (rr + 1) % RB
                pltpu.async_copy(tbl.at[nbr_v.at[g + 1]], rows_v.at[nb],
                                 gsem.at[nb])

            for c in range(CH):
                sl = pl.ds(c * 16, 16)
                ridx = jnp.full((16,), c * 16, jnp.int32) + iota
                plsc.addupdate_scatter(
                    rowbuf, [ridx, jnp.zeros((16,), jnp.int32)], lcx_v[g, sl])
                plsc.addupdate_scatter(
                    rowbuf, [ridx, jnp.full((16,), 1, jnp.int32)], lcy_v[g, sl])
                plsc.addupdate_scatter(
                    rowbuf, [ridx, jnp.full((16,), 2, jnp.int32)], lcz_v[g, sl])
            pltpu.make_async_copy(zero_v, grid_sh.at[sid, rr],
                                  zsem.at[rr]).wait()
            pltpu.sync_copy(rowbuf, grid_sh.at[sid, rr].at[sbin_v.at[g]],
                            add=True)
            gg = wid * (GRP_W // 4) + go
            pltpu.async_copy(grid_sh.at[sid, rr, pl.ds(0, GU)],
                             feat.at[gg, pl.ds(0, GU), 2 * rr], osem.at[rr])
            pltpu.async_copy(grid_sh.at[sid, rr, pl.ds(GU, GU)],
                             feat.at[gg, pl.ds(0, GU), 2 * rr + 1],
                             osem.at[rr])

            @pl.when((g >= 2) & (g + 2 < GRP_W))
            def _():
                rn = (rr + 2) % GB
                ggp = wid * (GRP_W // 4) + (g - 2) // 4
                pltpu.make_async_copy(grid_sh.at[sid, rn, pl.ds(0, GU)],
                                      feat.at[ggp, pl.ds(0, GU), 2 * rn],
                                      osem.at[rn]).wait()
                pltpu.make_async_copy(grid_sh.at[sid, rn, pl.ds(GU, GU)],
                                      feat.at[ggp, pl.ds(0, GU), 2 * rn + 1],
                                      osem.at[rn]).wait()
                pltpu.async_copy(zero_v, grid_sh.at[sid, rn], zsem.at[rn])
        return 0

    lax.fori_loop(0, GRP_W // GB, _quad_body, 0)
    gg_last = wid * (GRP_W // 4) + (GRP_W // 4) - 1
    for r in range(GB):
        pltpu.make_async_copy(grid_sh.at[sid, r, pl.ds(0, GU)],
                              feat.at[gg_last, pl.ds(0, GU), 2 * r],
                              osem.at[r]).wait()
        pltpu.make_async_copy(grid_sh.at[sid, r, pl.ds(GU, GU)],
                              feat.at[gg_last, pl.ds(0, GU), 2 * r + 1],
                              osem.at[r]).wait()


def _sc_build_feat(h, tbl, nbrr, lcx, lcy, lcz, didx, xyztbl):
    mesh = plsc.VectorSubcoreMesh(core_axis_name="c", subcore_axis_name="s")
    return pl.kernel(
        functools.partial(_sc_kernel_body, h),
        out_type=(
            jax.ShapeDtypeStruct((HP // 8, GU, 8, IN_CH), jnp.float32),
            jax.ShapeDtypeStruct((XROWS, GW, 128), jnp.float32),
        ),
        mesh=mesh,
        compiler_params=pltpu.CompilerParams(needs_layout_passes=False,
                                             use_tc_tiling_on_sc=False),
        scratch_types=(
            pltpu.VMEM((XROW_W, GW), jnp.int32),
            pltpu.VMEM((GRP_W, GW), jnp.int32),
            pltpu.VMEM((GRP_W, GW), jnp.int32),
            pltpu.VMEM((GRP_W, GW), jnp.float32),
            pltpu.VMEM((GRP_W, GW), jnp.float32),
            pltpu.VMEM((GRP_W, GW), jnp.float32),
            pltpu.VMEM((RB, GW, IN_CH), jnp.float32),
            pltpu.VMEM((PG * GU, IN_CH), jnp.float32),
            pltpu.VMEM_SHARED((16, GB, PG * GU, IN_CH), jnp.float32),
            pltpu.SemaphoreType.DMA,
            pltpu.SemaphoreType.DMA((RB,)),
            pltpu.SemaphoreType.DMA((GB,)),
            pltpu.SemaphoreType.DMA((GB,)),
        ),
    )(tbl, nbrr, lcx, lcy, lcz, didx, xyztbl)


ROWS_BLK = 512


def _tc_matmul_body(feat_ref, wv_ref, wg_ref, bias_ref, out_ref):
    wv = wv_ref[...]
    nrm2 = jnp.sum(wv * wv, axis=1, keepdims=True)
    scale = wg_ref[...] * lax.rsqrt(nrm2)
    wvs = wv[:, : GU * IN_CH] * scale
    acc = lax.dot_general(
        feat_ref[...].astype(jnp.bfloat16), wvs.astype(jnp.bfloat16),
        (((1,), (1,)), ((), ())),
        preferred_element_type=jnp.float32)
    out_ref[...] = jnp.maximum(acc + bias_ref[...], 0.0)


def _tc_matmul(feat2d, weight_v, weight_g, bias_row):
    grid = (HP // ROWS_BLK,)
    return pl.pallas_call(
        _tc_matmul_body,
        grid=grid,
        in_specs=[
            pl.BlockSpec((ROWS_BLK, GU * IN_CH), lambda i: (i, 0)),
            pl.BlockSpec((OUT_CH, G * IN_CH), lambda i: (0, 0)),
            pl.BlockSpec((OUT_CH, 1), lambda i: (0, 0)),
            pl.BlockSpec((1, OUT_CH), lambda i: (0, 0)),
        ],
        out_specs=pl.BlockSpec((ROWS_BLK, OUT_CH), lambda i: (i, 0)),
        out_shape=jax.ShapeDtypeStruct((HP, OUT_CH), jnp.float32),
    )(feat2d, weight_v, weight_g, bias_row)


def kernel(xyz, points, local_coordinates, neighbor_lists, parameter_list,
           data_idx, weight_v, weight_g, bias):
    del parameter_list
    pts = points.astype(jnp.float32).reshape(B * N, IN_CH - 3)
    tbl = jnp.pad(pts, ((0, 0), (3, 0)))
    nbrr = neighbor_lists.astype(jnp.int32).reshape(NGRP, GW)
    lc = local_coordinates.astype(jnp.float32)
    lcx = lc[..., 0].reshape(NGRP, GW)
    lcy = lc[..., 1].reshape(NGRP, GW)
    lcz = lc[..., 2].reshape(NGRP, GW)
    didx = data_idx.astype(jnp.int32).reshape(NH * XROWS, GW)
    xyztbl = jnp.pad(xyz.astype(jnp.float32).reshape(B * N, 3),
                     ((0, 0), (0, 125)))

    wv = weight_v.astype(jnp.float32)
    wg = weight_g.astype(jnp.float32).reshape(OUT_CH, 1)
    br = bias.astype(jnp.float32).reshape(1, OUT_CH)

    outs, nxyzs = [], []
    for h in range(NH):
        hs = pl.ds(h * HNGRP, HNGRP)
        feat, nxyz = _sc_build_feat(
            h, tbl, nbrr[h * HNGRP:(h + 1) * HNGRP],
            lcx[h * HNGRP:(h + 1) * HNGRP], lcy[h * HNGRP:(h + 1) * HNGRP],
            lcz[h * HNGRP:(h + 1) * HNGRP],
            didx[h * XROWS:(h + 1) * XROWS], xyztbl)
        del hs
        outs.append(_tc_matmul(
            feat.transpose(0, 2, 1, 3).reshape(HP, GU * IN_CH), wv, wg, br))
        nxyzs.append(nxyz.reshape(HP, 128)[:, :3])

    out = jnp.concatenate(outs, axis=0)
    new_xyz = jnp.concatenate(nxyzs, axis=0).reshape(B, NPOINT, 3)
    return new_xyz, out.reshape(B, NPOINT, OUT_CH)


# speedup vs baseline: 1.1114x; 1.1114x over previous
import functools

import jax
import jax.numpy as jnp
from jax import lax
from jax.experimental import pallas as pl
from jax.experimental.pallas import tpu as pltpu
from jax.experimental.pallas import tpu_sc as plsc

B, N, NPOINT, K = 4, 8192, 2048, 32
IN_CH, OUT_CH = 128, 256
PART, RADIUS = 5, 2.0
G = PART * PART + 1
GU = 20
P = B * NPOINT
PG = 2
NGRP = P // PG
GW = PG * K
CH = GW // 16
NW = 32
GRP_W = NGRP // NW
XROWS = P // GW
XROW_W = XROWS // NW


RB = 4
GB = 4


def _sc_kernel_body(tbl, nbrr, lcx, lcy, lcz, didx, xyztbl,
                    feat, nxyz,
                    didx_v, nbr_v, sbin_v,
                    lcx_v, lcy_v, lcz_v, rows_v, zero_v,
                    grid_sh, sem, gsem, zsem, osem):
    cid = lax.axis_index("c")
    sid = lax.axis_index("s")
    wid = cid * 16 + sid
    bn = (wid // (NW // B)) * N
    iota = lax.iota(jnp.int32, 16)

    pltpu.sync_copy(didx.at[pl.ds(wid * XROW_W, XROW_W)], didx_v)
    for r in range(XROW_W):
        row = wid * XROW_W + r
        for c in range(GW // 16):
            sl = pl.ds(c * 16, 16)
            didx_v[r, sl] = didx_v[r, sl] + bn
        pltpu.async_copy(xyztbl.at[didx_v.at[r]], rows_v.at[0], sem).wait()
        pltpu.sync_copy(rows_v.at[0], nxyz.at[row])

    g0 = wid * GRP_W
    pltpu.sync_copy(nbrr.at[pl.ds(g0, GRP_W)], nbr_v)
    pltpu.sync_copy(lcx.at[pl.ds(g0, GRP_W)], lcx_v)
    pltpu.sync_copy(lcy.at[pl.ds(g0, GRP_W)], lcy_v)
    pltpu.sync_copy(lcz.at[pl.ds(g0, GRP_W)], lcz_v)

    zv = jnp.zeros((16,), jnp.float32)

    def _zero_body(i, _):
        for c in range(8):
            zero_v[i, pl.ds(c * 16, 16)] = zv
        return 0

    lax.fori_loop(0, PG * GU, _zero_body, 0)

    def _idx_body(g, _):
        for c in range(CH):
            sl = pl.ds(c * 16, 16)
            nbr_v[g, sl] = nbr_v[g, sl] + bn
            x = lcx_v[g, sl]
            y = lcy_v[g, sl]
            fx = (x + RADIUS) / (2.0 * RADIUS) * (PART - 1 - 0.01)
            fy = (y + RADIUS) / (2.0 * RADIUS) * (PART - 1 - 0.01)
            bidx = fx.astype(jnp.int32) * PART + fy.astype(jnp.int32) + 1
            filt = ((x * x > RADIUS * RADIUS) |
                    (y * y > RADIUS * RADIUS))
            sbin_v[g, sl] = jnp.where(filt, 0, bidx) + (c // 2) * GU
        return 0

    lax.fori_loop(0, GRP_W, _idx_body, 0)

    for r in range(GB):
        pltpu.async_copy(zero_v, grid_sh.at[sid, r], zsem.at[r])
    pltpu.async_copy(tbl.at[nbr_v.at[0]], rows_v.at[0], gsem.at[0])

    def _quad_body(go, _):
        for rr in range(GB):
            g = go * GB + rr
            rb = rr % RB
            rowbuf = rows_v.at[rb]
            pltpu.make_async_copy(tbl.at[nbr_v.at[g]], rowbuf,
                                  gsem.at[rb]).wait()

            @pl.when(g + 1 < GRP_W)
            def _():
                nb = (rr + 1) % RB
                pltpu.async_copy(tbl.at[nbr_v.at[g + 1]], rows_v.at[nb],
                                 gsem.at[nb])

            for c in range(CH):
                sl = pl.ds(c * 16, 16)
                ridx = jnp.full((16,), c * 16, jnp.int32) + iota
                plsc.addupdate_scatter(
                    rowbuf, [ridx, jnp.zeros((16,), jnp.int32)], lcx_v[g, sl])
                plsc.addupdate_scatter(
                    rowbuf, [ridx, jnp.full((16,), 1, jnp.int32)], lcy_v[g, sl])
                plsc.addupdate_scatter(
                    rowbuf, [ridx, jnp.full((16,), 2, jnp.int32)], lcz_v[g, sl])
            pltpu.make_async_copy(zero_v, grid_sh.at[sid, rr],
                                  zsem.at[rr]).wait()
            pltpu.sync_copy(rowbuf, grid_sh.at[sid, rr].at[sbin_v.at[g]],
                            add=True)
            gg = wid * (GRP_W // 4) + go
            pltpu.async_copy(grid_sh.at[sid, rr, pl.ds(0, GU)],
                             feat.at[gg, pl.ds(0, GU), 2 * rr], osem.at[rr])
            pltpu.async_copy(grid_sh.at[sid, rr, pl.ds(GU, GU)],
                             feat.at[gg, pl.ds(0, GU), 2 * rr + 1],
                             osem.at[rr])

            @pl.when((g >= 2) & (g + 2 < GRP_W))
            def _():
                rn = (rr + 2) % GB
                ggp = wid * (GRP_W // 4) + (g - 2) // 4
                pltpu.make_async_copy(grid_sh.at[sid, rn, pl.ds(0, GU)],
                                      feat.at[ggp, pl.ds(0, GU), 2 * rn],
                                      osem.at[rn]).wait()
                pltpu.make_async_copy(grid_sh.at[sid, rn, pl.ds(GU, GU)],
                                      feat.at[ggp, pl.ds(0, GU), 2 * rn + 1],
                                      osem.at[rn]).wait()
                pltpu.async_copy(zero_v, grid_sh.at[sid, rn], zsem.at[rn])
        return 0

    lax.fori_loop(0, GRP_W // GB, _quad_body, 0)
    gg_last = wid * (GRP_W // 4) + (GRP_W // 4) - 1
    for r in range(GB):
        pltpu.make_async_copy(grid_sh.at[sid, r, pl.ds(0, GU)],
                              feat.at[gg_last, pl.ds(0, GU), 2 * r],
                              osem.at[r]).wait()
        pltpu.make_async_copy(grid_sh.at[sid, r, pl.ds(GU, GU)],
                              feat.at[gg_last, pl.ds(0, GU), 2 * r + 1],
                              osem.at[r]).wait()


def _sc_build_feat(tbl, nbrr, lcx, lcy, lcz, didx, xyztbl):
    mesh = plsc.VectorSubcoreMesh(core_axis_name="c", subcore_axis_name="s")
    return pl.kernel(
        _sc_kernel_body,
        out_type=(
            jax.ShapeDtypeStruct((P // 8, GU, 8, IN_CH), jnp.float32),
            jax.ShapeDtypeStruct((XROWS, GW, 128), jnp.float32),
        ),
        mesh=mesh,
        compiler_params=pltpu.CompilerParams(needs_layout_passes=False,
                                             use_tc_tiling_on_sc=False),
        scratch_types=(
            pltpu.VMEM((XROW_W, GW), jnp.int32),
            pltpu.VMEM((GRP_W, GW), jnp.int32),
            pltpu.VMEM((GRP_W, GW), jnp.int32),
            pltpu.VMEM((GRP_W, GW), jnp.float32),
            pltpu.VMEM((GRP_W, GW), jnp.float32),
            pltpu.VMEM((GRP_W, GW), jnp.float32),
            pltpu.VMEM((RB, GW, IN_CH), jnp.float32),
            pltpu.VMEM((PG * GU, IN_CH), jnp.float32),
            pltpu.VMEM_SHARED((16, GB, PG * GU, IN_CH), jnp.float32),
            pltpu.SemaphoreType.DMA,
            pltpu.SemaphoreType.DMA((RB,)),
            pltpu.SemaphoreType.DMA((GB,)),
            pltpu.SemaphoreType.DMA((GB,)),
        ),
    )(tbl, nbrr, lcx, lcy, lcz, didx, xyztbl)


ROWS_BLK = 512


def _tc_matmul_body(feat_ref, wv_ref, wg_ref, bias_ref, out_ref):
    wv = wv_ref[...]
    nrm2 = jnp.sum(wv * wv, axis=1, keepdims=True)
    scale = wg_ref[...] * lax.rsqrt(nrm2)
    wvs = wv[:, : GU * IN_CH] * scale
    acc = lax.dot_general(
        feat_ref[...].astype(jnp.bfloat16), wvs.astype(jnp.bfloat16),
        (((1,), (1,)), ((), ())),
        preferred_element_type=jnp.float32)
    out_ref[...] = jnp.maximum(acc + bias_ref[...], 0.0)


def _tc_matmul(feat2d, weight_v, weight_g, bias_row):
    grid = (P // ROWS_BLK,)
    return pl.pallas_call(
        _tc_matmul_body,
        grid=grid,
        in_specs=[
            pl.BlockSpec((ROWS_BLK, GU * IN_CH), lambda i: (i, 0)),
            pl.BlockSpec((OUT_CH, G * IN_CH), lambda i: (0, 0)),
            pl.BlockSpec((OUT_CH, 1), lambda i: (0, 0)),
            pl.BlockSpec((1, OUT_CH), lambda i: (0, 0)),
        ],
        out_specs=pl.BlockSpec((ROWS_BLK, OUT_CH), lambda i: (i, 0)),
        out_shape=jax.ShapeDtypeStruct((P, OUT_CH), jnp.float32),
    )(feat2d, weight_v, weight_g, bias_row)


def kernel(xyz, points, local_coordinates, neighbor_lists, parameter_list,
           data_idx, weight_v, weight_g, bias):
    del parameter_list
    pts = points.astype(jnp.float32).reshape(B * N, IN_CH - 3)
    tbl = jnp.pad(pts, ((0, 0), (3, 0)))
    nbrr = neighbor_lists.astype(jnp.int32).reshape(NGRP, GW)
    lc = local_coordinates.astype(jnp.float32)
    lcx = lc[..., 0].reshape(NGRP, GW)
    lcy = lc[..., 1].reshape(NGRP, GW)
    lcz = lc[..., 2].reshape(NGRP, GW)
    didx = data_idx.astype(jnp.int32).reshape(XROWS, GW)
    xyztbl = jnp.pad(xyz.astype(jnp.float32).reshape(B * N, 3),
                     ((0, 0), (0, 125)))

    feat, nxyz = _sc_build_feat(tbl, nbrr, lcx, lcy, lcz, didx, xyztbl)

    out = _tc_matmul(feat.transpose(0, 2, 1, 3).reshape(P, GU * IN_CH),
                     weight_v.astype(jnp.float32),
                     weight_g.astype(jnp.float32).reshape(OUT_CH, 1),
                     bias.astype(jnp.float32).reshape(1, OUT_CH))

    new_xyz = nxyz.reshape(P, 128)[:, :3].reshape(B, NPOINT, 3)
    return new_xyz, out.reshape(B, NPOINT, OUT_CH)
